# Initial kernel scaffold; baseline (speedup 1.0000x reference)
import jax
import jax.numpy as jnp
from jax.experimental import pallas as pl


def kernel(x, edge_index, W, b):
    raise NotImplementedError("write your pallas kernel here")


# speedup vs baseline: 3.3451x; 3.3451x over previous
import functools

import jax
import jax.numpy as jnp
from jax import lax
from jax.experimental import pallas as pl
from jax.experimental.pallas import tpu as pltpu
from jax.experimental.pallas import tpu_sc as plsc

N_NODES = 10000
N_EDGES = 320000
NFEAT = 128
NHID = 128

NC = 2
NS = 16
NW = NC * NS
L = 16

CHUNK = 128
N_CHUNKS = 80
E_PER_W = N_CHUNKS * CHUNK
E_PAD = NW * E_PER_W
PAD_ROW = N_NODES
ACC_ROWS = 10240
ROWS_PER_TILE = ACC_ROWS // NS


def _matmul_body(x_ref, w_ref, b_ref, o_ref):
    o_ref[...] = (
        jnp.dot(x_ref[...], w_ref[...], preferred_element_type=jnp.float32)
        + b_ref[...]
    )


def _support_matmul(x, W, b):
    B = 1000
    return pl.pallas_call(
        _matmul_body,
        grid=(N_NODES // B,),
        in_specs=[
            pl.BlockSpec((B, NFEAT), lambda i: (i, 0)),
            pl.BlockSpec((NFEAT, NHID), lambda i: (0, 0)),
            pl.BlockSpec((1, NHID), lambda i: (0, 0)),
        ],
        out_specs=pl.BlockSpec((B, NHID), lambda i: (i, 0)),
        out_shape=jax.ShapeDtypeStruct((N_NODES, NHID), jnp.float32),
    )(x, W, b.reshape(1, NHID))


def _sc_body(support_hbm, src_hbm, dst_hbm, part_hbm,
             sidx_v, didx_v, rows_v, acc_sh, sem):
    cid = lax.axis_index("c")
    sid = lax.axis_index("s")
    wid = sid * NC + cid

    pltpu.sync_copy(src_hbm.at[wid], sidx_v)
    pltpu.sync_copy(dst_hbm.at[wid], didx_v)

    zeros = jnp.zeros((L,), jnp.float32)

    def _zero_row(i, _):
        for j in range(NHID // L):
            rows_v[i, pl.ds(j * L, L)] = zeros
        return 0

    lax.fori_loop(0, CHUNK, _zero_row, 0)
    for k in range(ROWS_PER_TILE // CHUNK):
        pltpu.sync_copy(
            rows_v, acc_sh.at[pl.ds(sid * ROWS_PER_TILE + k * CHUNK, CHUNK)]
        )
    plsc.subcore_barrier()

    def _chunk_body(c, _):
        pltpu.async_copy(support_hbm.at[sidx_v.at[c]], rows_v, sem).wait()
        pltpu.sync_copy(rows_v, acc_sh.at[didx_v.at[c]], add=True)
        return 0

    lax.fori_loop(0, N_CHUNKS, _chunk_body, 0)
    plsc.subcore_barrier()

    for k in range(ROWS_PER_TILE // CHUNK):
        r0 = sid * ROWS_PER_TILE + k * CHUNK
        pltpu.sync_copy(acc_sh.at[pl.ds(r0, CHUNK)], rows_v)
        pltpu.sync_copy(rows_v, part_hbm.at[cid, pl.ds(r0, CHUNK)])


_sc_scatter = functools.partial(
    pl.kernel,
    out_type=jax.ShapeDtypeStruct((NC, ACC_ROWS, NHID), jnp.float32),
    mesh=plsc.VectorSubcoreMesh(core_axis_name="c", subcore_axis_name="s"),
    scratch_types=[
        pltpu.VMEM((N_CHUNKS, CHUNK), jnp.int32),
        pltpu.VMEM((N_CHUNKS, CHUNK), jnp.int32),
        pltpu.VMEM((CHUNK, NHID), jnp.float32),
        pltpu.VMEM_SHARED((ACC_ROWS, NHID), jnp.float32),
        pltpu.SemaphoreType.DMA,
    ],
)(_sc_body)


def _combine_body(p0_ref, p1_ref, o_ref):
    o_ref[...] = jnp.maximum(p0_ref[0] + p1_ref[0], 0.0)


def _combine(part):
    B = 1000
    return pl.pallas_call(
        _combine_body,
        grid=(N_NODES // B,),
        in_specs=[
            pl.BlockSpec((1, B, NHID), lambda i: (0, i, 0)),
            pl.BlockSpec((1, B, NHID), lambda i: (1, i, 0)),
        ],
        out_specs=pl.BlockSpec((B, NHID), lambda i: (i, 0)),
        out_shape=jax.ShapeDtypeStruct((N_NODES, NHID), jnp.float32),
    )(part, part)


def kernel(x, edge_index, W, b):
    support = _support_matmul(x, W, b)

    n_pad = E_PAD - N_EDGES
    src = jnp.concatenate(
        [edge_index[0].astype(jnp.int32), jnp.zeros((n_pad,), jnp.int32)]
    ).reshape(NW, N_CHUNKS, CHUNK)
    dst = jnp.concatenate(
        [edge_index[1].astype(jnp.int32),
         jnp.full((n_pad,), PAD_ROW, jnp.int32)]
    ).reshape(NW, N_CHUNKS, CHUNK)

    part = _sc_scatter(support, src, dst)
    return _combine(part)


# speedup vs baseline: 3.5768x; 1.0693x over previous
import functools

import jax
import jax.numpy as jnp
from jax import lax
from jax.experimental import pallas as pl
from jax.experimental.pallas import tpu as pltpu
from jax.experimental.pallas import tpu_sc as plsc

N_NODES = 10000
N_EDGES = 320000
NFEAT = 128
NHID = 128

NC = 2
NS = 16
NW = NC * NS
L = 16

CHUNK = 128
N_CHUNKS = 80
E_PER_W = N_CHUNKS * CHUNK
E_PAD = NW * E_PER_W
PAD_ROW = N_NODES
ACC_ROWS = 10240
ROWS_PER_TILE = ACC_ROWS // NS


def _matmul_body(x_ref, w_ref, b_ref, o_ref):
    o_ref[...] = (
        jnp.dot(x_ref[...], w_ref[...], preferred_element_type=jnp.float32)
        + b_ref[...]
    )


def _support_matmul(x, W, b):
    B = 1000
    return pl.pallas_call(
        _matmul_body,
        grid=(N_NODES // B,),
        in_specs=[
            pl.BlockSpec((B, NFEAT), lambda i: (i, 0)),
            pl.BlockSpec((NFEAT, NHID), lambda i: (0, 0)),
            pl.BlockSpec((1, NHID), lambda i: (0, 0)),
        ],
        out_specs=pl.BlockSpec((B, NHID), lambda i: (i, 0)),
        out_shape=jax.ShapeDtypeStruct((N_NODES, NHID), jnp.float32),
    )(x, W, b.reshape(1, NHID))


NBUF = 2
HC = N_CHUNKS // 2


def _sc_body(support_hbm, src_hbm, dst_hbm, part_hbm,
             sidx_v, didx_v, rows0, rows1,
             acc_sh, gs0, gs1, ss0, ss1):
    rows = [rows0, rows1]
    gsem = [gs0, gs1]
    ssem = [ss0, ss1]
    cid = lax.axis_index("c")
    sid = lax.axis_index("s")
    wid = sid * NC + cid

    zeros = jnp.zeros((L,), jnp.float32)

    def _zero_row(i, _):
        for j in range(NHID // L):
            rows0[i, pl.ds(j * L, L)] = zeros
        return 0

    lax.fori_loop(0, CHUNK, _zero_row, 0)
    for k in range(ROWS_PER_TILE // CHUNK):
        pltpu.sync_copy(
            rows0, acc_sh.at[pl.ds(sid * ROWS_PER_TILE + k * CHUNK, CHUNK)]
        )
    plsc.subcore_barrier()

    for h in range(2):
        pltpu.sync_copy(src_hbm.at[wid, pl.ds(h * HC, HC)], sidx_v)
        pltpu.sync_copy(dst_hbm.at[wid, pl.ds(h * HC, HC)], didx_v)

        pltpu.async_copy(support_hbm.at[sidx_v.at[0]], rows0, gs0)

        def _round(r, _):
            for b in range(NBUF):
                c = r * NBUF + b
                bn = 1 - b

                @pl.when(c >= 1)
                def _():
                    pltpu.make_async_copy(
                        rows[bn], acc_sh.at[didx_v.at[c - 1]], ssem[bn]
                    ).wait()

                @pl.when(c + 1 < HC)
                def _():
                    pltpu.async_copy(
                        support_hbm.at[sidx_v.at[c + 1]], rows[bn], gsem[bn]
                    )

                pltpu.make_async_copy(
                    support_hbm.at[sidx_v.at[c]], rows[b], gsem[b]
                ).wait()
                pltpu.async_copy(
                    rows[b], acc_sh.at[didx_v.at[c]], ssem[b], add=True
                )
            return 0

        lax.fori_loop(0, HC // NBUF, _round, 0)
        pltpu.make_async_copy(
            rows1, acc_sh.at[didx_v.at[HC - 1]], ss1
        ).wait()
    plsc.subcore_barrier()

    for k in range(ROWS_PER_TILE // CHUNK):
        r0 = sid * ROWS_PER_TILE + k * CHUNK
        pltpu.sync_copy(acc_sh.at[pl.ds(r0, CHUNK)], rows0)
        pltpu.sync_copy(rows0, part_hbm.at[cid, pl.ds(r0, CHUNK)])


_sc_scatter = functools.partial(
    pl.kernel,
    out_type=jax.ShapeDtypeStruct((NC, ACC_ROWS, NHID), jnp.float32),
    mesh=plsc.VectorSubcoreMesh(core_axis_name="c", subcore_axis_name="s"),
    scratch_types=[
        pltpu.VMEM((HC, CHUNK), jnp.int32),
        pltpu.VMEM((HC, CHUNK), jnp.int32),
    ] + [pltpu.VMEM((CHUNK, NHID), jnp.float32)] * NBUF
    + [pltpu.VMEM_SHARED((ACC_ROWS, NHID), jnp.float32)]
    + [pltpu.SemaphoreType.DMA] * (2 * NBUF),
)(_sc_body)


def _combine_body(p0_ref, p1_ref, o_ref):
    o_ref[...] = jnp.maximum(p0_ref[0] + p1_ref[0], 0.0)


def _combine(part):
    B = 1000
    return pl.pallas_call(
        _combine_body,
        grid=(N_NODES // B,),
        in_specs=[
            pl.BlockSpec((1, B, NHID), lambda i: (0, i, 0)),
            pl.BlockSpec((1, B, NHID), lambda i: (1, i, 0)),
        ],
        out_specs=pl.BlockSpec((B, NHID), lambda i: (i, 0)),
        out_shape=jax.ShapeDtypeStruct((N_NODES, NHID), jnp.float32),
    )(part, part)


def kernel(x, edge_index, W, b):
    support = _support_matmul(x, W, b)

    n_pad = E_PAD - N_EDGES
    src = jnp.concatenate(
        [edge_index[0].astype(jnp.int32), jnp.zeros((n_pad,), jnp.int32)]
    ).reshape(NW, N_CHUNKS, CHUNK)
    dst = jnp.concatenate(
        [edge_index[1].astype(jnp.int32),
         jnp.full((n_pad,), PAD_ROW, jnp.int32)]
    ).reshape(NW, N_CHUNKS, CHUNK)

    part = _sc_scatter(support, src, dst)
    return _combine(part)


# speedup vs baseline: 3.7348x; 1.0442x over previous
import functools

import jax
import jax.numpy as jnp
from jax import lax
from jax.experimental import pallas as pl
from jax.experimental.pallas import tpu as pltpu
from jax.experimental.pallas import tpu_sc as plsc

N_NODES = 10000
N_EDGES = 320000
NFEAT = 128
NHID = 128

NC = 2
NS = 16
NW = NC * NS
L = 16

CHUNK = 64
N_CHUNKS = 160
E_PER_W = N_CHUNKS * CHUNK
E_PAD = NW * E_PER_W
PAD_ROW = N_NODES
ACC_ROWS = 10240
ROWS_PER_TILE = ACC_ROWS // NS


def _matmul_body(x_ref, w_ref, b_ref, o_ref):
    o_ref[...] = (
        jnp.dot(x_ref[...], w_ref[...], preferred_element_type=jnp.float32)
        + b_ref[...]
    )


def _support_matmul(x, W, b):
    B = 1000
    return pl.pallas_call(
        _matmul_body,
        grid=(N_NODES // B,),
        in_specs=[
            pl.BlockSpec((B, NFEAT), lambda i: (i, 0)),
            pl.BlockSpec((NFEAT, NHID), lambda i: (0, 0)),
            pl.BlockSpec((1, NHID), lambda i: (0, 0)),
        ],
        out_specs=pl.BlockSpec((B, NHID), lambda i: (i, 0)),
        out_shape=jax.ShapeDtypeStruct((N_NODES, NHID), jnp.float32),
    )(x, W, b.reshape(1, NHID))


NBUF = 4
LA = 2
HC = N_CHUNKS // 4


def _sc_body(support_hbm, src_hbm, dst_hbm, part_hbm,
             sidx_v, didx_v, rows0, rows1, rows2, rows3,
             acc_sh, gs0, gs1, gs2, gs3, ss0, ss1, ss2, ss3):
    rows = [rows0, rows1, rows2, rows3]
    gsem = [gs0, gs1, gs2, gs3]
    ssem = [ss0, ss1, ss2, ss3]
    cid = lax.axis_index("c")
    sid = lax.axis_index("s")
    wid = sid * NC + cid

    zeros = jnp.zeros((L,), jnp.float32)

    def _zero_row(i, _):
        for j in range(NHID // L):
            rows0[i, pl.ds(j * L, L)] = zeros
        return 0

    lax.fori_loop(0, CHUNK, _zero_row, 0)
    for k in range(ROWS_PER_TILE // CHUNK):
        pltpu.sync_copy(
            rows0, acc_sh.at[pl.ds(sid * ROWS_PER_TILE + k * CHUNK, CHUNK)]
        )
    plsc.subcore_barrier()

    for h in range(4):
        pltpu.sync_copy(src_hbm.at[wid, pl.ds(h * HC, HC)], sidx_v)
        pltpu.sync_copy(dst_hbm.at[wid, pl.ds(h * HC, HC)], didx_v)

        for j in range(LA):
            pltpu.async_copy(support_hbm.at[sidx_v.at[j]], rows[j], gsem[j])

        def _round(r, _):
            for b in range(NBUF):
                c = r * NBUF + b
                bn = (b + LA) % NBUF

                @pl.when(c >= NBUF - LA)
                def _():
                    pltpu.make_async_copy(
                        rows[bn], acc_sh.at[didx_v.at[c - (NBUF - LA)]],
                        ssem[bn],
                    ).wait()

                @pl.when(c + LA < HC)
                def _():
                    pltpu.async_copy(
                        support_hbm.at[sidx_v.at[c + LA]], rows[bn], gsem[bn]
                    )

                pltpu.make_async_copy(
                    support_hbm.at[sidx_v.at[c]], rows[b], gsem[b]
                ).wait()
                pltpu.async_copy(
                    rows[b], acc_sh.at[didx_v.at[c]], ssem[b], add=True
                )
            return 0

        lax.fori_loop(0, HC // NBUF, _round, 0)
        for j in range(NBUF - LA):
            b = (HC - (NBUF - LA) + j) % NBUF
            pltpu.make_async_copy(
                rows[b], acc_sh.at[didx_v.at[HC - 1]], ssem[b]
            ).wait()
    plsc.subcore_barrier()

    for k in range(ROWS_PER_TILE // CHUNK):
        r0 = sid * ROWS_PER_TILE + k * CHUNK
        pltpu.sync_copy(acc_sh.at[pl.ds(r0, CHUNK)], rows0)
        pltpu.sync_copy(rows0, part_hbm.at[cid, pl.ds(r0, CHUNK)])


_sc_scatter = functools.partial(
    pl.kernel,
    out_type=jax.ShapeDtypeStruct((NC, ACC_ROWS, NHID), jnp.float32),
    mesh=plsc.VectorSubcoreMesh(core_axis_name="c", subcore_axis_name="s"),
    scratch_types=[
        pltpu.VMEM((HC, CHUNK), jnp.int32),
        pltpu.VMEM((HC, CHUNK), jnp.int32),
    ] + [pltpu.VMEM((CHUNK, NHID), jnp.float32)] * NBUF
    + [pltpu.VMEM_SHARED((ACC_ROWS, NHID), jnp.float32)]
    + [pltpu.SemaphoreType.DMA] * (2 * NBUF),
)(_sc_body)


def _combine_body(p0_ref, p1_ref, o_ref):
    o_ref[...] = jnp.maximum(p0_ref[0] + p1_ref[0], 0.0)


def _combine(part):
    B = 1000
    return pl.pallas_call(
        _combine_body,
        grid=(N_NODES // B,),
        in_specs=[
            pl.BlockSpec((1, B, NHID), lambda i: (0, i, 0)),
            pl.BlockSpec((1, B, NHID), lambda i: (1, i, 0)),
        ],
        out_specs=pl.BlockSpec((B, NHID), lambda i: (i, 0)),
        out_shape=jax.ShapeDtypeStruct((N_NODES, NHID), jnp.float32),
    )(part, part)


def kernel(x, edge_index, W, b):
    support = _support_matmul(x, W, b)

    n_pad = E_PAD - N_EDGES
    src = jnp.concatenate(
        [edge_index[0].astype(jnp.int32), jnp.zeros((n_pad,), jnp.int32)]
    ).reshape(NW, N_CHUNKS, CHUNK)
    dst = jnp.concatenate(
        [edge_index[1].astype(jnp.int32),
         jnp.full((n_pad,), PAD_ROW, jnp.int32)]
    ).reshape(NW, N_CHUNKS, CHUNK)

    part = _sc_scatter(support, src, dst)
    return _combine(part)


# speedup vs baseline: 7.9000x; 2.1152x over previous
import functools

import jax
import jax.numpy as jnp
from jax import lax
from jax.experimental import pallas as pl
from jax.experimental.pallas import tpu as pltpu
from jax.experimental.pallas import tpu_sc as plsc

N_NODES = 10000
N_EDGES = 320000
NFEAT = 128
NHID = 128

NC = 2
NS = 16
NW = NC * NS
L = 16

CHUNK = 64
N_CHUNKS = 160
E_PER_W = N_CHUNKS * CHUNK
E_PAD = NW * E_PER_W
PAD_ROW = N_NODES
TAB_ROWS = 10240
ROWS_PER_TILE = TAB_ROWS // NS


def _matmul_body(x_ref, w_ref, b_ref, o_ref):
    o_ref[...] = (
        jnp.dot(x_ref[...], w_ref[...], preferred_element_type=jnp.float32)
        + b_ref[...]
    )


def _support_matmul(xp, W, b):
    B = 1024
    return pl.pallas_call(
        _matmul_body,
        grid=(TAB_ROWS // B,),
        in_specs=[
            pl.BlockSpec((B, NFEAT), lambda i: (i, 0)),
            pl.BlockSpec((NFEAT, NHID), lambda i: (0, 0)),
            pl.BlockSpec((1, NHID), lambda i: (0, 0)),
        ],
        out_specs=pl.BlockSpec((B, NHID), lambda i: (i, 0)),
        out_shape=jax.ShapeDtypeStruct((TAB_ROWS, NHID), jnp.float32),
    )(xp, W, b.reshape(1, NHID))


NBUF = 4
LA = 2
HC = N_CHUNKS // 4


def _sc_body(sup_hbm, src_hbm, dst_hbm, part_hbm, msgs_hbm,
             sidx_v, didx_v, rows0, rows1, rows2, rows3,
             sh, gs0, gs1, gs2, gs3, ws0, ws1, ws2, ws3):
    rows = [rows0, rows1, rows2, rows3]
    gsem = [gs0, gs1, gs2, gs3]
    wsem = [ws0, ws1, ws2, ws3]
    cid = lax.axis_index("c")
    sid = lax.axis_index("s")
    wid = sid * NC + cid
    ebase = wid * E_PER_W

    for k in range(ROWS_PER_TILE // CHUNK):
        r0 = sid * ROWS_PER_TILE + k * CHUNK
        pltpu.sync_copy(sup_hbm.at[pl.ds(r0, CHUNK)], rows0)
        pltpu.sync_copy(rows0, sh.at[pl.ds(r0, CHUNK)])
    plsc.subcore_barrier()

    for h in range(4):
        pltpu.sync_copy(src_hbm.at[wid, pl.ds(h * HC, HC)], sidx_v)

        for j in range(LA):
            pltpu.async_copy(sh.at[sidx_v.at[j]], rows[j], gsem[j])

        def _round1(r, _):
            for b in range(NBUF):
                c = r * NBUF + b
                bn = (b + LA) % NBUF

                @pl.when(c >= NBUF - LA)
                def _():
                    cc = c - (NBUF - LA)
                    pltpu.make_async_copy(
                        rows[bn],
                        msgs_hbm.at[pl.ds(ebase + (h * HC + cc) * CHUNK,
                                          CHUNK)],
                        wsem[bn],
                    ).wait()

                @pl.when(c + LA < HC)
                def _():
                    pltpu.async_copy(
                        sh.at[sidx_v.at[c + LA]], rows[bn], gsem[bn]
                    )

                pltpu.make_async_copy(
                    sh.at[sidx_v.at[c]], rows[b], gsem[b]
                ).wait()
                pltpu.async_copy(
                    rows[b],
                    msgs_hbm.at[pl.ds(ebase + (h * HC + c) * CHUNK, CHUNK)],
                    wsem[b],
                )
            return 0

        lax.fori_loop(0, HC // NBUF, _round1, 0)
        for j in range(NBUF - LA):
            cc = HC - (NBUF - LA) + j
            b = cc % NBUF
            pltpu.make_async_copy(
                rows[b],
                msgs_hbm.at[pl.ds(ebase + (h * HC + cc) * CHUNK, CHUNK)],
                wsem[b],
            ).wait()
    plsc.subcore_barrier()

    zeros = jnp.zeros((L,), jnp.float32)

    def _zero_row(i, _):
        for j in range(NHID // L):
            rows0[i, pl.ds(j * L, L)] = zeros
        return 0

    lax.fori_loop(0, CHUNK, _zero_row, 0)
    for k in range(ROWS_PER_TILE // CHUNK):
        r0 = sid * ROWS_PER_TILE + k * CHUNK
        pltpu.sync_copy(rows0, sh.at[pl.ds(r0, CHUNK)])
    plsc.subcore_barrier()

    for h in range(4):
        pltpu.sync_copy(dst_hbm.at[wid, pl.ds(h * HC, HC)], didx_v)

        for j in range(LA):
            pltpu.async_copy(
                msgs_hbm.at[pl.ds(ebase + (h * HC + j) * CHUNK, CHUNK)],
                rows[j], gsem[j],
            )

        def _round2(r, _):
            for b in range(NBUF):
                c = r * NBUF + b
                bn = (b + LA) % NBUF

                @pl.when(c >= NBUF - LA)
                def _():
                    pltpu.make_async_copy(
                        rows[bn], sh.at[didx_v.at[c - (NBUF - LA)]],
                        wsem[bn],
                    ).wait()

                @pl.when(c + LA < HC)
                def _():
                    pltpu.async_copy(
                        msgs_hbm.at[pl.ds(ebase + (h * HC + c + LA) * CHUNK,
                                          CHUNK)],
                        rows[bn], gsem[bn],
                    )

                pltpu.make_async_copy(
                    msgs_hbm.at[pl.ds(ebase + (h * HC + c) * CHUNK, CHUNK)],
                    rows[b], gsem[b],
                ).wait()
                pltpu.async_copy(
                    rows[b], sh.at[didx_v.at[c]], wsem[b], add=True
                )
            return 0

        lax.fori_loop(0, HC // NBUF, _round2, 0)
        for j in range(NBUF - LA):
            b = (HC - (NBUF - LA) + j) % NBUF
            pltpu.make_async_copy(
                rows[b], sh.at[didx_v.at[HC - 1]], wsem[b]
            ).wait()
    plsc.subcore_barrier()

    for k in range(ROWS_PER_TILE // CHUNK):
        r0 = sid * ROWS_PER_TILE + k * CHUNK
        pltpu.sync_copy(sh.at[pl.ds(r0, CHUNK)], rows0)
        pltpu.sync_copy(rows0, part_hbm.at[cid, pl.ds(r0, CHUNK)])


_sc_scatter = functools.partial(
    pl.kernel,
    out_type=[
        jax.ShapeDtypeStruct((NC, TAB_ROWS, NHID), jnp.float32),
        jax.ShapeDtypeStruct((E_PAD, NHID), jnp.float32),
    ],
    mesh=plsc.VectorSubcoreMesh(core_axis_name="c", subcore_axis_name="s"),
    scratch_types=[
        pltpu.VMEM((HC, CHUNK), jnp.int32),
        pltpu.VMEM((HC, CHUNK), jnp.int32),
    ] + [pltpu.VMEM((CHUNK, NHID), jnp.float32)] * NBUF
    + [pltpu.VMEM_SHARED((TAB_ROWS, NHID), jnp.float32)]
    + [pltpu.SemaphoreType.DMA] * (2 * NBUF),
)(_sc_body)


def _combine_body(p0_ref, p1_ref, o_ref):
    o_ref[...] = jnp.maximum(p0_ref[0] + p1_ref[0], 0.0)


def _combine(part):
    B = 1000
    return pl.pallas_call(
        _combine_body,
        grid=(N_NODES // B,),
        in_specs=[
            pl.BlockSpec((1, B, NHID), lambda i: (0, i, 0)),
            pl.BlockSpec((1, B, NHID), lambda i: (1, i, 0)),
        ],
        out_specs=pl.BlockSpec((B, NHID), lambda i: (i, 0)),
        out_shape=jax.ShapeDtypeStruct((N_NODES, NHID), jnp.float32),
    )(part, part)


def kernel(x, edge_index, W, b):
    xp = jnp.pad(x, ((0, TAB_ROWS - N_NODES), (0, 0)))
    support = _support_matmul(xp, W, b)

    n_pad = E_PAD - N_EDGES
    src = jnp.concatenate(
        [edge_index[0].astype(jnp.int32), jnp.zeros((n_pad,), jnp.int32)]
    ).reshape(NW, N_CHUNKS, CHUNK)
    dst = jnp.concatenate(
        [edge_index[1].astype(jnp.int32),
         jnp.full((n_pad,), PAD_ROW, jnp.int32)]
    ).reshape(NW, N_CHUNKS, CHUNK)

    part, _ = _sc_scatter(support, src, dst)
    return _combine(part)


# speedup vs baseline: 8.2988x; 1.0505x over previous
import functools

import jax
import jax.numpy as jnp
from jax import lax
from jax.experimental import pallas as pl
from jax.experimental.pallas import tpu as pltpu
from jax.experimental.pallas import tpu_sc as plsc

N_NODES = 10000
N_EDGES = 320000
NFEAT = 128
NHID = 128

NC = 2
NS = 16
NW = NC * NS
L = 16

CHUNK = 128
N_CHUNKS = 80
E_PER_W = N_CHUNKS * CHUNK
E_PAD = NW * E_PER_W
PAD_ROW = N_NODES
TAB_ROWS = 10240
ROWS_PER_TILE = TAB_ROWS // NS


def _matmul_body(x_ref, w_ref, b_ref, o_ref):
    o_ref[...] = (
        jnp.dot(x_ref[...], w_ref[...], preferred_element_type=jnp.float32)
        + b_ref[...]
    )


def _support_matmul(xp, W, b):
    B = 1024
    return pl.pallas_call(
        _matmul_body,
        grid=(TAB_ROWS // B,),
        in_specs=[
            pl.BlockSpec((B, NFEAT), lambda i: (i, 0)),
            pl.BlockSpec((NFEAT, NHID), lambda i: (0, 0)),
            pl.BlockSpec((1, NHID), lambda i: (0, 0)),
        ],
        out_specs=pl.BlockSpec((B, NHID), lambda i: (i, 0)),
        out_shape=jax.ShapeDtypeStruct((TAB_ROWS, NHID), jnp.float32),
    )(xp, W, b.reshape(1, NHID))


NBUF = 2
LA = 1
HC = N_CHUNKS // 2


def _sc_body(sup_hbm, src_hbm, dst_hbm, part_hbm, msgs_hbm,
             sidx_v, didx_v, rows0, rows1,
             sh, gs0, gs1, ws0, ws1):
    rows = [rows0, rows1]
    gsem = [gs0, gs1]
    wsem = [ws0, ws1]
    cid = lax.axis_index("c")
    sid = lax.axis_index("s")
    wid = sid * NC + cid
    ebase = wid * E_PER_W

    for k in range(ROWS_PER_TILE // CHUNK):
        r0 = sid * ROWS_PER_TILE + k * CHUNK
        pltpu.sync_copy(sup_hbm.at[pl.ds(r0, CHUNK)], rows0)
        pltpu.sync_copy(rows0, sh.at[pl.ds(r0, CHUNK)])
    plsc.subcore_barrier()

    for h in range(2):
        pltpu.sync_copy(src_hbm.at[wid, pl.ds(h * HC, HC)], sidx_v)

        for j in range(LA):
            pltpu.async_copy(sh.at[sidx_v.at[j]], rows[j], gsem[j])

        def _round1(r, _):
            for b in range(NBUF):
                c = r * NBUF + b
                bn = (b + LA) % NBUF

                @pl.when(c >= NBUF - LA)
                def _():
                    cc = c - (NBUF - LA)
                    pltpu.make_async_copy(
                        rows[bn],
                        msgs_hbm.at[pl.ds(ebase + (h * HC + cc) * CHUNK,
                                          CHUNK)],
                        wsem[bn],
                    ).wait()

                @pl.when(c + LA < HC)
                def _():
                    pltpu.async_copy(
                        sh.at[sidx_v.at[c + LA]], rows[bn], gsem[bn]
                    )

                pltpu.make_async_copy(
                    sh.at[sidx_v.at[c]], rows[b], gsem[b]
                ).wait()
                pltpu.async_copy(
                    rows[b],
                    msgs_hbm.at[pl.ds(ebase + (h * HC + c) * CHUNK, CHUNK)],
                    wsem[b],
                )
            return 0

        lax.fori_loop(0, HC // NBUF, _round1, 0)
        for j in range(NBUF - LA):
            cc = HC - (NBUF - LA) + j
            b = cc % NBUF
            pltpu.make_async_copy(
                rows[b],
                msgs_hbm.at[pl.ds(ebase + (h * HC + cc) * CHUNK, CHUNK)],
                wsem[b],
            ).wait()
    plsc.subcore_barrier()

    zeros = jnp.zeros((L,), jnp.float32)

    def _zero_row(i, _):
        for j in range(NHID // L):
            rows0[i, pl.ds(j * L, L)] = zeros
        return 0

    lax.fori_loop(0, CHUNK, _zero_row, 0)
    for k in range(ROWS_PER_TILE // CHUNK):
        r0 = sid * ROWS_PER_TILE + k * CHUNK
        pltpu.sync_copy(rows0, sh.at[pl.ds(r0, CHUNK)])
    plsc.subcore_barrier()

    for h in range(2):
        pltpu.sync_copy(dst_hbm.at[wid, pl.ds(h * HC, HC)], didx_v)

        for j in range(LA):
            pltpu.async_copy(
                msgs_hbm.at[pl.ds(ebase + (h * HC + j) * CHUNK, CHUNK)],
                rows[j], gsem[j],
            )

        def _round2(r, _):
            for b in range(NBUF):
                c = r * NBUF + b
                bn = (b + LA) % NBUF

                @pl.when(c >= NBUF - LA)
                def _():
                    pltpu.make_async_copy(
                        rows[bn], sh.at[didx_v.at[c - (NBUF - LA)]],
                        wsem[bn],
                    ).wait()

                @pl.when(c + LA < HC)
                def _():
                    pltpu.async_copy(
                        msgs_hbm.at[pl.ds(ebase + (h * HC + c + LA) * CHUNK,
                                          CHUNK)],
                        rows[bn], gsem[bn],
                    )

                pltpu.make_async_copy(
                    msgs_hbm.at[pl.ds(ebase + (h * HC + c) * CHUNK, CHUNK)],
                    rows[b], gsem[b],
                ).wait()
                pltpu.async_copy(
                    rows[b], sh.at[didx_v.at[c]], wsem[b], add=True
                )
            return 0

        lax.fori_loop(0, HC // NBUF, _round2, 0)
        for j in range(NBUF - LA):
            b = (HC - (NBUF - LA) + j) % NBUF
            pltpu.make_async_copy(
                rows[b], sh.at[didx_v.at[HC - 1]], wsem[b]
            ).wait()
    plsc.subcore_barrier()

    for k in range(ROWS_PER_TILE // CHUNK):
        r0 = sid * ROWS_PER_TILE + k * CHUNK
        pltpu.sync_copy(sh.at[pl.ds(r0, CHUNK)], rows0)
        pltpu.sync_copy(rows0, part_hbm.at[cid, pl.ds(r0, CHUNK)])


_sc_scatter = functools.partial(
    pl.kernel,
    out_type=[
        jax.ShapeDtypeStruct((NC, TAB_ROWS, NHID), jnp.float32),
        jax.ShapeDtypeStruct((E_PAD, NHID), jnp.float32),
    ],
    mesh=plsc.VectorSubcoreMesh(core_axis_name="c", subcore_axis_name="s"),
    scratch_types=[
        pltpu.VMEM((HC, CHUNK), jnp.int32),
        pltpu.VMEM((HC, CHUNK), jnp.int32),
    ] + [pltpu.VMEM((CHUNK, NHID), jnp.float32)] * NBUF
    + [pltpu.VMEM_SHARED((TAB_ROWS, NHID), jnp.float32)]
    + [pltpu.SemaphoreType.DMA] * (2 * NBUF),
)(_sc_body)


def _combine_body(p0_ref, p1_ref, o_ref):
    o_ref[...] = jnp.maximum(p0_ref[0] + p1_ref[0], 0.0)


def _combine(part):
    B = 1000
    return pl.pallas_call(
        _combine_body,
        grid=(N_NODES // B,),
        in_specs=[
            pl.BlockSpec((1, B, NHID), lambda i: (0, i, 0)),
            pl.BlockSpec((1, B, NHID), lambda i: (1, i, 0)),
        ],
        out_specs=pl.BlockSpec((B, NHID), lambda i: (i, 0)),
        out_shape=jax.ShapeDtypeStruct((N_NODES, NHID), jnp.float32),
    )(part, part)


def kernel(x, edge_index, W, b):
    xp = jnp.pad(x, ((0, TAB_ROWS - N_NODES), (0, 0)))
    support = _support_matmul(xp, W, b)

    n_pad = E_PAD - N_EDGES
    src = jnp.concatenate(
        [edge_index[0].astype(jnp.int32), jnp.zeros((n_pad,), jnp.int32)]
    ).reshape(NW, N_CHUNKS, CHUNK)
    dst = jnp.concatenate(
        [edge_index[1].astype(jnp.int32),
         jnp.full((n_pad,), PAD_ROW, jnp.int32)]
    ).reshape(NW, N_CHUNKS, CHUNK)

    part, _ = _sc_scatter(support, src, dst)
    return _combine(part)
